# Initial kernel scaffold; baseline (speedup 1.0000x reference)
import jax
import jax.numpy as jnp
from jax.experimental import pallas as pl


def kernel(x, edge_index, W_conv, b_conv, ln_g, ln_b, ln2_g, ln2_b, W_att, a_src, a_dst, W_proj, b_proj):
    raise NotImplementedError("write your pallas kernel here")


# speedup vs baseline: 15.8626x; 15.8626x over previous
import functools

import jax
import jax.numpy as jnp
from jax import lax
from jax.experimental import pallas as pl
from jax.experimental.pallas import tpu as pltpu
from jax.experimental.pallas import tpu_sc as plsc

N = 10000
D = 128
H = 4
NCONV = 2

NCORE = 2
NSUB = 16
NW = NCORE * NSUB
LANES = 16
CHUNK = 128
NP = 10240
ROWS_PER_SUB = NP // NSUB

_mesh = plsc.VectorSubcoreMesh(core_axis_name="c", subcore_axis_name="s")
_sc_params = pltpu.CompilerParams(needs_layout_passes=False)


def _pad_edge_arrays(src, dst):
    e = src.shape[0]
    nch = -(-e // (NW * CHUNK))
    ep = NW * CHUNK * nch
    pad = ep - e
    src_p = jnp.concatenate([src, jnp.full((pad,), N, jnp.int32)])
    dst_p = jnp.concatenate([dst, jnp.full((pad,), N, jnp.int32)])
    return (src_p.reshape(NW, nch, CHUNK), dst_p.reshape(NW, nch, CHUNK), nch)


def _sc_degree(dst_p, nch):
    @functools.partial(
        pl.kernel, mesh=_mesh, compiler_params=_sc_params,
        out_type=jax.ShapeDtypeStruct((NW, NP), jnp.float32),
        scratch_types=[
            pltpu.VMEM((nch, CHUNK), jnp.int32),
            pltpu.VMEM((NP,), jnp.float32),
            pltpu.SemaphoreType.DMA,
        ],
    )
    def k(dst_hbm, out_hbm, idx_v, deg_v, sem):
        c = lax.axis_index("c")
        s = lax.axis_index("s")
        w = s * NCORE + c
        pltpu.sync_copy(dst_hbm.at[w], idx_v)

        @pl.loop(0, NP, step=LANES)
        def _zero(i):
            deg_v[pl.ds(i, LANES)] = jnp.zeros((LANES,), jnp.float32)

        ones = jnp.ones((LANES,), jnp.float32)

        @pl.loop(0, nch)
        def _edges(j):
            for q in range(CHUNK // LANES):
                dvec = idx_v[j, pl.ds(q * LANES, LANES)]
                plsc.addupdate_scatter(deg_v, [dvec], ones)

        pltpu.sync_copy(deg_v, out_hbm.at[w])

    return k(dst_p)


def _sc_segsum(h_pad, src_p, dst_p, nch):
    @functools.partial(
        pl.kernel, mesh=_mesh, compiler_params=_sc_params,
        out_type=jax.ShapeDtypeStruct((NCORE, NSUB, ROWS_PER_SUB, D),
                                      jnp.float32),
        scratch_types=[
            pltpu.VMEM((nch, CHUNK), jnp.int32),
            pltpu.VMEM((nch, CHUNK), jnp.int32),
            pltpu.VMEM((CHUNK, D), jnp.float32),
            pltpu.VMEM_SHARED((NP, D), jnp.float32),
            pltpu.SemaphoreType.DMA,
        ],
    )
    def k(h_hbm, src_hbm, dst_hbm, out_hbm, src_v, dst_v, rows_v,
          acc_sh, sem):
        c = lax.axis_index("c")
        s = lax.axis_index("s")
        w = s * NCORE + c
        pltpu.sync_copy(src_hbm.at[w], src_v)
        pltpu.sync_copy(dst_hbm.at[w], dst_v)

        @pl.loop(0, CHUNK)
        def _zb(i):
            for q in range(D // LANES):
                rows_v[i, pl.ds(q * LANES, LANES)] = jnp.zeros((LANES,),
                                                               jnp.float32)

        base = s * ROWS_PER_SUB

        @pl.loop(0, ROWS_PER_SUB, step=CHUNK)
        def _za(r):
            pltpu.sync_copy(rows_v, acc_sh.at[pl.ds(base + r, CHUNK)])

        plsc.subcore_barrier()

        @pl.loop(0, nch)
        def _edges(j):
            pltpu.sync_copy(h_hbm.at[src_v.at[j]], rows_v)
            pltpu.sync_copy(rows_v, acc_sh.at[dst_v.at[j]], add=True)

        plsc.subcore_barrier()
        pltpu.sync_copy(acc_sh.at[pl.ds(base, ROWS_PER_SUB)], out_hbm.at[c, s])

    out = k(h_pad, src_p, dst_p)
    return out.reshape(NCORE, NP, D)


def _sc_attention(z_pad, es_pad, ed_pad, src_p, dst_p, nch):
    @functools.partial(
        pl.kernel, mesh=_mesh, compiler_params=_sc_params,
        out_type=(
            jax.ShapeDtypeStruct((NCORE, NSUB, ROWS_PER_SUB, D), jnp.float32),
            jax.ShapeDtypeStruct((NW, NP), jnp.float32),
        ),
        scratch_types=[
            pltpu.VMEM((2, CHUNK), jnp.int32),
            pltpu.VMEM((2, CHUNK), jnp.int32),
            pltpu.VMEM((NP,), jnp.float32),
            pltpu.VMEM((NP,), jnp.float32),
            pltpu.VMEM((NP,), jnp.float32),
            pltpu.VMEM((CHUNK,), jnp.float32),
            pltpu.VMEM((CHUNK, D), jnp.float32),
            pltpu.VMEM_SHARED((NP, D), jnp.float32),
            pltpu.SemaphoreType.DMA,
        ],
    )
    def k(z_hbm, es_hbm, ed_hbm, src_hbm, dst_hbm, num_hbm, den_hbm,
          src_v, dst_v, es_v, ed_v, den_v, w_v, rows_v, acc_sh, sem):
        c = lax.axis_index("c")
        s = lax.axis_index("s")
        w = s * NCORE + c
        pltpu.sync_copy(es_hbm, es_v)
        pltpu.sync_copy(ed_hbm, ed_v)

        @pl.loop(0, NP, step=LANES)
        def _zd(i):
            den_v[pl.ds(i, LANES)] = jnp.zeros((LANES,), jnp.float32)

        @pl.loop(0, CHUNK)
        def _zb(i):
            for q in range(D // LANES):
                rows_v[i, pl.ds(q * LANES, LANES)] = jnp.zeros((LANES,),
                                                               jnp.float32)

        base = s * ROWS_PER_SUB

        @pl.loop(0, ROWS_PER_SUB, step=CHUNK)
        def _za(r):
            pltpu.sync_copy(rows_v, acc_sh.at[pl.ds(base + r, CHUNK)])

        plsc.subcore_barrier()

        @pl.loop(0, nch)
        def _edges(j):
            pltpu.sync_copy(src_hbm.at[w, j], src_v.at[0])
            pltpu.sync_copy(dst_hbm.at[w, j], dst_v.at[0])
            pltpu.sync_copy(z_hbm.at[src_v.at[0]], rows_v)
            for q in range(CHUNK // LANES):
                svec = src_v[0, pl.ds(q * LANES, LANES)]
                dvec = dst_v[0, pl.ds(q * LANES, LANES)]
                sv = plsc.load_gather(es_v, [svec])
                dv = plsc.load_gather(ed_v, [dvec])
                e = sv + dv
                e = jnp.where(e > 0, e, 0.2 * e)
                ex = jnp.exp(e)
                w_v[pl.ds(q * LANES, LANES)] = ex
                plsc.addupdate_scatter(den_v, [dvec], ex)

            @pl.loop(0, CHUNK)
            def _scale(ei):
                wb = plsc.load_gather(w_v, [jnp.broadcast_to(ei, (LANES,))])
                for q in range(D // LANES):
                    sl = pl.ds(q * LANES, LANES)
                    rows_v[ei, sl] = rows_v[ei, sl] * wb

            pltpu.sync_copy(rows_v, acc_sh.at[dst_v.at[0]], add=True)

        plsc.subcore_barrier()
        pltpu.sync_copy(acc_sh.at[pl.ds(base, ROWS_PER_SUB)], num_hbm.at[c, s])
        pltpu.sync_copy(den_v, den_hbm.at[w])

    num, den = k(z_pad, es_pad, ed_pad, src_p, dst_p)
    return num.reshape(NCORE, NP, D), den


def kernel(x, edge_index, W_conv, b_conv, ln_g, ln_b, ln2_g, ln2_b, W_att,
           a_src, a_dst, W_proj, b_proj):
    src = edge_index[0]
    dst = edge_index[1]
    src_p, dst_p, nch = _pad_edge_arrays(src, dst)

    deg_part = _sc_degree(dst_p, nch)
    deg = jnp.sum(deg_part, axis=0)[:N] + 1.0
    rs = lax.rsqrt(deg)
    rs_pad = jnp.pad(rs, (0, NP - N))

    xp = jnp.pad(x, ((0, NP - N), (0, 0)))
    for i in range(NCONV):
        h = (xp[:N] @ W_conv[i] + b_conv[i]) * rs[:, None]
        h_pad = jnp.pad(h, ((0, NP - N), (0, 0)))
        p = _sc_segsum(h_pad, src_p, dst_p, nch)
        t = (p[0, :N] + p[1, :N]) * rs[:, None]
        mu = jnp.mean(t, axis=-1, keepdims=True)
        v = jnp.var(t, axis=-1, keepdims=True)
        t = (t - mu) / jnp.sqrt(v + 1e-5) * ln_g + ln_b
        t = jax.nn.relu(t)
        xp = jnp.pad(t, ((0, NP - N), (0, 0)))

    heads = []
    for hh in range(H):
        z = xp[:N] @ W_att[hh]
        es = z @ a_src[hh]
        ed = z @ a_dst[hh]
        z_pad = jnp.pad(z, ((0, NP - N), (0, 0)))
        es_pad = jnp.pad(es, (0, NP - N))
        ed_pad = jnp.pad(ed, (0, NP - N))
        num, den = _sc_attention(z_pad, es_pad, ed_pad, src_p, dst_p, nch)
        numer = num[0, :N] + num[1, :N]
        den_t = jnp.sum(den, axis=0)[:N] + 1e-9
        heads.append(numer / den_t[:, None])
    xa = jnp.concatenate(heads, axis=-1)
    mu = jnp.mean(xa, axis=-1, keepdims=True)
    v = jnp.var(xa, axis=-1, keepdims=True)
    xa = (xa - mu) / jnp.sqrt(v + 1e-5) * ln2_g + ln2_b
    return jax.nn.relu(xa @ W_proj + b_proj)


# speedup vs baseline: 16.1265x; 1.0166x over previous
import functools

import jax
import jax.numpy as jnp
from jax import lax
from jax.experimental import pallas as pl
from jax.experimental.pallas import tpu as pltpu
from jax.experimental.pallas import tpu_sc as plsc

N = 10000
D = 128
H = 4
NCONV = 2

NCORE = 2
NSUB = 16
NW = NCORE * NSUB
LANES = 16
CHUNK = 128
NP = 10240
ROWS_PER_SUB = NP // NSUB

_mesh = plsc.VectorSubcoreMesh(core_axis_name="c", subcore_axis_name="s")
_sc_params = pltpu.CompilerParams(needs_layout_passes=False)


def _pad_edge_arrays(src, dst):
    e = src.shape[0]
    nch = -(-e // (NW * CHUNK))
    ep = NW * CHUNK * nch
    pad = ep - e
    src_p = jnp.concatenate([src, jnp.full((pad,), N, jnp.int32)])
    dst_p = jnp.concatenate([dst, jnp.full((pad,), N, jnp.int32)])
    return (src_p.reshape(NW, nch, CHUNK), dst_p.reshape(NW, nch, CHUNK), nch)


def _sc_degree(dst_p, nch):
    @functools.partial(
        pl.kernel, mesh=_mesh, compiler_params=_sc_params,
        out_type=jax.ShapeDtypeStruct((NW, NP), jnp.float32),
        scratch_types=[
            pltpu.VMEM((nch, CHUNK), jnp.int32),
            pltpu.VMEM((NP,), jnp.float32),
            pltpu.SemaphoreType.DMA,
        ],
    )
    def k(dst_hbm, out_hbm, idx_v, deg_v, sem):
        c = lax.axis_index("c")
        s = lax.axis_index("s")
        w = s * NCORE + c
        pltpu.sync_copy(dst_hbm.at[w], idx_v)

        @pl.loop(0, NP, step=LANES)
        def _zero(i):
            deg_v[pl.ds(i, LANES)] = jnp.zeros((LANES,), jnp.float32)

        ones = jnp.ones((LANES,), jnp.float32)

        @pl.loop(0, nch)
        def _edges(j):
            for q in range(CHUNK // LANES):
                dvec = idx_v[j, pl.ds(q * LANES, LANES)]
                plsc.addupdate_scatter(deg_v, [dvec], ones)

        pltpu.sync_copy(deg_v, out_hbm.at[w])

    return k(dst_p)


def _sc_segsum(h_pad, src_p, dst_p, nch):
    @functools.partial(
        pl.kernel, mesh=_mesh, compiler_params=_sc_params,
        out_type=jax.ShapeDtypeStruct((NCORE, NSUB, ROWS_PER_SUB, D),
                                      jnp.float32),
        scratch_types=[
            pltpu.VMEM((nch, CHUNK), jnp.int32),
            pltpu.VMEM((nch, CHUNK), jnp.int32),
            pltpu.VMEM((CHUNK, D), jnp.float32),
            pltpu.VMEM_SHARED((NP, D), jnp.float32),
            pltpu.SemaphoreType.DMA,
        ],
    )
    def k(h_hbm, src_hbm, dst_hbm, out_hbm, src_v, dst_v, rows_v,
          acc_sh, sem):
        c = lax.axis_index("c")
        s = lax.axis_index("s")
        w = s * NCORE + c
        pltpu.sync_copy(src_hbm.at[w], src_v)
        pltpu.sync_copy(dst_hbm.at[w], dst_v)

        @pl.loop(0, CHUNK)
        def _zb(i):
            for q in range(D // LANES):
                rows_v[i, pl.ds(q * LANES, LANES)] = jnp.zeros((LANES,),
                                                               jnp.float32)

        base = s * ROWS_PER_SUB

        @pl.loop(0, ROWS_PER_SUB, step=CHUNK)
        def _za(r):
            pltpu.sync_copy(rows_v, acc_sh.at[pl.ds(base + r, CHUNK)])

        plsc.subcore_barrier()

        @pl.loop(0, nch)
        def _edges(j):
            pltpu.sync_copy(h_hbm.at[src_v.at[j]], rows_v)
            pltpu.sync_copy(rows_v, acc_sh.at[dst_v.at[j]], add=True)

        plsc.subcore_barrier()
        pltpu.sync_copy(acc_sh.at[pl.ds(base, ROWS_PER_SUB)], out_hbm.at[c, s])

    out = k(h_pad, src_p, dst_p)
    return out.reshape(NCORE, NP, D)


def _sc_attention(z_pad, es_pad, ed_pad, src_p, dst_p, nch):
    @functools.partial(
        pl.kernel, mesh=_mesh, compiler_params=_sc_params,
        out_type=(
            jax.ShapeDtypeStruct((NCORE, NSUB, ROWS_PER_SUB, D), jnp.float32),
            jax.ShapeDtypeStruct((NW, NP), jnp.float32),
        ),
        scratch_types=[
            pltpu.VMEM((2, CHUNK), jnp.int32),
            pltpu.VMEM((2, CHUNK), jnp.int32),
            pltpu.VMEM((NP,), jnp.float32),
            pltpu.VMEM((NP,), jnp.float32),
            pltpu.VMEM((NP,), jnp.float32),
            pltpu.VMEM((CHUNK,), jnp.float32),
            pltpu.VMEM((CHUNK, D), jnp.float32),
            pltpu.VMEM_SHARED((NP, D), jnp.float32),
            pltpu.SemaphoreType.DMA,
        ],
    )
    def k(z_hbm, es_hbm, ed_hbm, src_hbm, dst_hbm, num_hbm, den_hbm,
          src_v, dst_v, es_v, ed_v, den_v, w_v, rows_v, acc_sh, sem):
        c = lax.axis_index("c")
        s = lax.axis_index("s")
        w = s * NCORE + c
        pltpu.sync_copy(es_hbm, es_v)
        pltpu.sync_copy(ed_hbm, ed_v)

        @pl.loop(0, NP, step=LANES)
        def _zd(i):
            den_v[pl.ds(i, LANES)] = jnp.zeros((LANES,), jnp.float32)

        @pl.loop(0, CHUNK)
        def _zb(i):
            for q in range(D // LANES):
                rows_v[i, pl.ds(q * LANES, LANES)] = jnp.zeros((LANES,),
                                                               jnp.float32)

        base = s * ROWS_PER_SUB

        @pl.loop(0, ROWS_PER_SUB, step=CHUNK)
        def _za(r):
            pltpu.sync_copy(rows_v, acc_sh.at[pl.ds(base + r, CHUNK)])

        plsc.subcore_barrier()

        @pl.loop(0, nch)
        def _edges(j):
            pltpu.sync_copy(src_hbm.at[w, j], src_v.at[0])
            pltpu.sync_copy(dst_hbm.at[w, j], dst_v.at[0])
            pltpu.sync_copy(z_hbm.at[src_v.at[0]], rows_v)
            for q in range(CHUNK // LANES):
                svec = src_v[0, pl.ds(q * LANES, LANES)]
                dvec = dst_v[0, pl.ds(q * LANES, LANES)]
                sv = plsc.load_gather(es_v, [svec])
                dv = plsc.load_gather(ed_v, [dvec])
                e = sv + dv
                e = jnp.where(e > 0, e, 0.2 * e)
                ex = jnp.exp(e)
                w_v[pl.ds(q * LANES, LANES)] = ex
                plsc.addupdate_scatter(den_v, [dvec], ex)

            @pl.loop(0, CHUNK)
            def _scale(ei):
                wb = plsc.load_gather(w_v, [jnp.broadcast_to(ei, (LANES,))])
                for q in range(D // LANES):
                    sl = pl.ds(q * LANES, LANES)
                    rows_v[ei, sl] = rows_v[ei, sl] * wb

            pltpu.sync_copy(rows_v, acc_sh.at[dst_v.at[0]], add=True)

        plsc.subcore_barrier()
        pltpu.sync_copy(acc_sh.at[pl.ds(base, ROWS_PER_SUB)], num_hbm.at[c, s])
        pltpu.sync_copy(den_v, den_hbm.at[w])

    num, den = k(z_pad, es_pad, ed_pad, src_p, dst_p)
    return num.reshape(NCORE, NP, D), den


BN = 2048


def _rows(pid):
    return pid * BN + lax.broadcasted_iota(jnp.int32, (BN, 1), 0)


def _ln_rows(t, g, b):
    mu = jnp.mean(t, axis=-1, keepdims=True)
    v = jnp.mean((t - mu) ** 2, axis=-1, keepdims=True)
    return (t - mu) / jnp.sqrt(v + 1e-5) * g + b


def _tc_rs(deg_t):
    def body(d_ref, o_ref):
        o_ref[...] = lax.rsqrt(
            jnp.sum(d_ref[...], axis=1, keepdims=True) + 1.0)

    return pl.pallas_call(
        body, out_shape=jax.ShapeDtypeStruct((NP, 1), jnp.float32))(deg_t)


def _tc_prescale(xp, W, b, rs):
    def body(x_ref, w_ref, b_ref, rs_ref, o_ref):
        h = jnp.dot(x_ref[...], w_ref[...],
                    preferred_element_type=jnp.float32) + b_ref[...]
        h = h * rs_ref[...]
        o_ref[...] = jnp.where(_rows(pl.program_id(0)) < N, h, 0.0)

    return pl.pallas_call(
        body,
        grid=(NP // BN,),
        in_specs=[pl.BlockSpec((BN, D), lambda i: (i, 0)),
                  pl.BlockSpec((D, D), lambda i: (0, 0)),
                  pl.BlockSpec((1, D), lambda i: (0, 0)),
                  pl.BlockSpec((BN, 1), lambda i: (i, 0))],
        out_specs=pl.BlockSpec((BN, D), lambda i: (i, 0)),
        out_shape=jax.ShapeDtypeStruct((NP, D), jnp.float32),
    )(xp, W, b.reshape(1, D), rs)


def _tc_conv_mid(p, rs, g, bb, W, b):
    def body(p_ref, rs_ref, g_ref, bb_ref, w_ref, b_ref, o_ref):
        rsb = rs_ref[...]
        t = (p_ref[0] + p_ref[1]) * rsb
        y = jax.nn.relu(_ln_rows(t, g_ref[...], bb_ref[...]))
        h = (jnp.dot(y, w_ref[...], preferred_element_type=jnp.float32)
             + b_ref[...]) * rsb
        o_ref[...] = jnp.where(_rows(pl.program_id(0)) < N, h, 0.0)

    return pl.pallas_call(
        body,
        grid=(NP // BN,),
        in_specs=[pl.BlockSpec((NCORE, BN, D), lambda i: (0, i, 0)),
                  pl.BlockSpec((BN, 1), lambda i: (i, 0)),
                  pl.BlockSpec((1, D), lambda i: (0, 0)),
                  pl.BlockSpec((1, D), lambda i: (0, 0)),
                  pl.BlockSpec((D, D), lambda i: (0, 0)),
                  pl.BlockSpec((1, D), lambda i: (0, 0))],
        out_specs=pl.BlockSpec((BN, D), lambda i: (i, 0)),
        out_shape=jax.ShapeDtypeStruct((NP, D), jnp.float32),
    )(p, rs, g.reshape(1, D), bb.reshape(1, D), W, b.reshape(1, D))


def _tc_att_prep(p, rs, g, bb, W_att, a_cat):
    def body(p_ref, rs_ref, g_ref, bb_ref, watt_ref, a_ref,
             z0_ref, z1_ref, z2_ref, z3_ref, e8_ref):
        t = (p_ref[0] + p_ref[1]) * rs_ref[...]
        y = jax.nn.relu(_ln_rows(t, g_ref[...], bb_ref[...]))
        y = jnp.where(_rows(pl.program_id(0)) < N, y, 0.0)
        z_refs = [z0_ref, z1_ref, z2_ref, z3_ref]
        cols = []
        for h in range(H):
            z = jnp.dot(y, watt_ref[h], preferred_element_type=jnp.float32)
            z_refs[h][...] = z
            cols.append(jnp.dot(z, a_ref[h],
                                preferred_element_type=jnp.float32))
        e8_ref[...] = jnp.concatenate(cols, axis=-1)

    zs = jax.ShapeDtypeStruct((NP, D), jnp.float32)
    return pl.pallas_call(
        body,
        grid=(NP // BN,),
        in_specs=[pl.BlockSpec((NCORE, BN, D), lambda i: (0, i, 0)),
                  pl.BlockSpec((BN, 1), lambda i: (i, 0)),
                  pl.BlockSpec((1, D), lambda i: (0, 0)),
                  pl.BlockSpec((1, D), lambda i: (0, 0)),
                  pl.BlockSpec((H, D, D), lambda i: (0, 0, 0)),
                  pl.BlockSpec((H, D, 2), lambda i: (0, 0, 0))],
        out_specs=[pl.BlockSpec((BN, D), lambda i: (i, 0))] * H
        + [pl.BlockSpec((BN, 2 * H), lambda i: (i, 0))],
        out_shape=[zs] * H
        + [jax.ShapeDtypeStruct((NP, 2 * H), jnp.float32)],
    )(p, rs, g.reshape(1, D), bb.reshape(1, D), W_att, a_cat)


def _tc_final(nums, den_ts, g2, b2, Wp, bp):
    def body(n0, n1, n2, n3, d0, d1, d2, d3, g_ref, bb_ref, w_ref, b_ref,
             o_ref):
        hs = []
        for n_ref, d_ref in zip((n0, n1, n2, n3), (d0, d1, d2, d3)):
            den = jnp.sum(d_ref[...], axis=1, keepdims=True) + 1e-9
            hs.append((n_ref[0] + n_ref[1]) / den)
        hcat = jnp.concatenate(hs, axis=-1)
        y = _ln_rows(hcat, g_ref[...], bb_ref[...])
        y = jnp.dot(y, w_ref[...], preferred_element_type=jnp.float32)
        o_ref[...] = jax.nn.relu(y + b_ref[...])

    hd = 2 * H * D
    return pl.pallas_call(
        body,
        grid=(NP // BN,),
        in_specs=[pl.BlockSpec((NCORE, BN, D), lambda i: (0, i, 0))] * H
        + [pl.BlockSpec((BN, NW), lambda i: (i, 0))] * H
        + [pl.BlockSpec((1, H * D), lambda i: (0, 0)),
           pl.BlockSpec((1, H * D), lambda i: (0, 0)),
           pl.BlockSpec((H * D, D), lambda i: (0, 0)),
           pl.BlockSpec((1, D), lambda i: (0, 0))],
        out_specs=pl.BlockSpec((BN, D), lambda i: (i, 0)),
        out_shape=jax.ShapeDtypeStruct((NP, D), jnp.float32),
    )(*nums, *den_ts, g2.reshape(1, H * D), b2.reshape(1, H * D), Wp,
      bp.reshape(1, D))


def kernel(x, edge_index, W_conv, b_conv, ln_g, ln_b, ln2_g, ln2_b, W_att,
           a_src, a_dst, W_proj, b_proj):
    src = edge_index[0]
    dst = edge_index[1]
    src_p, dst_p, nch = _pad_edge_arrays(src, dst)

    deg_part = _sc_degree(dst_p, nch)
    rs = _tc_rs(deg_part.T)

    xp = jnp.pad(x, ((0, NP - N), (0, 0)))
    h = _tc_prescale(xp, W_conv[0], b_conv[0], rs)
    p = _sc_segsum(h, src_p, dst_p, nch)
    h = _tc_conv_mid(p, rs, ln_g, ln_b, W_conv[1], b_conv[1])
    p = _sc_segsum(h, src_p, dst_p, nch)

    a_cat = jnp.stack([a_src, a_dst], axis=-1)
    *zs, e8 = _tc_att_prep(p, rs, ln_g, ln_b, W_att, a_cat)

    nums, den_ts = [], []
    for hh in range(H):
        num, den = _sc_attention(zs[hh], e8[:, 2 * hh], e8[:, 2 * hh + 1],
                                 src_p, dst_p, nch)
        nums.append(num)
        den_ts.append(den.T)
    out = _tc_final(nums, den_ts, ln2_g, ln2_b, W_proj, b_proj)
    return out[:N]
